# baseline (speedup 1.0000x reference)
import jax
import jax.numpy as jnp
from jax import lax
from jax.experimental import pallas as pl
from jax.experimental.pallas import tpu as pltpu
from jax.experimental.pallas import tpu_sc as plsc

NC = 2
NS = 16
L = 16
NW = NC * NS
RB = 4
CB = 512
H_TC = 256
HC = 64


def _sc_body(x_hbm, part_hbm, buf0, buf1, acc, tot, sem0, sem1):
    B, C, H, W = x_hbm.shape
    w = lax.axis_index("s") * NC + lax.axis_index("c")
    b = w // 4
    rows_per_w = (H - H_TC) // 4
    row0 = H_TC + (w % 4) * rows_per_w
    n_cb = W // CB
    nchunk = (rows_per_w // RB) * n_cb

    zero = jnp.zeros((L,), jnp.float32)
    for i in range(C):
        acc[pl.ds(i * L, L)] = zero

    bufs = (buf0, buf1)
    sems = (sem0, sem1)
    lanes = lax.iota(jnp.int32, L)

    def copy(g, buf, sem):
        r = row0 + (g // n_cb) * RB
        col = (g % n_cb) * CB
        return pltpu.make_async_copy(
            x_hbm.at[b, :, pl.ds(r, RB), pl.ds(col, CB)], buf, sem)

    copy(0, buf0, sem0).start()

    def compute(buf):
        def combine(a, b_):
            ma, ia = a
            mb, ib = b_
            gt = mb > ma
            m = jnp.maximum(ma, mb)
            iav = jnp.full((L,), ia, jnp.int32) if isinstance(ia, int) else ia
            ibv = jnp.full((L,), ib, jnp.int32) if isinstance(ib, int) else ib
            return m, jnp.where(gt, ibv, iav)

        NV = CB // L
        def vbody(v, carry):
            i = v // NV
            s = (v % NV) * L
            nodes = [(buf[c, i, pl.ds(s, L)], c) for c in range(C)]
            while len(nodes) > 1:
                nxt = [combine(nodes[k], nodes[k + 1])
                       for k in range(0, len(nodes) - 1, 2)]
                if len(nodes) % 2:
                    nxt.append(nodes[-1])
                nodes = nxt
            m, idx = nodes[0]
            plsc.addupdate_scatter(acc, [idx * L + lanes], m)
            return carry

        lax.fori_loop(0, RB * CB // L, vbody, 0)

    for g in range(nchunk):
        copy(g, bufs[g % 2], sems[g % 2]).wait()
        if g + 1 < nchunk:
            copy(g + 1, bufs[(g + 1) % 2], sems[(g + 1) % 2]).start()
        compute(bufs[g % 2])

    ci1 = lanes
    ci2 = jnp.minimum(lanes + 16, C - 1)
    t1 = jnp.zeros((L,), jnp.float32)
    t2 = jnp.zeros((L,), jnp.float32)
    for l in range(L):
        t1 = t1 + plsc.load_gather(acc, [ci1 * L + l])
        t2 = t2 + plsc.load_gather(acc, [ci2 * L + l])
    t2 = jnp.where(lanes < C - 16, t2, 0.0)
    tot[pl.ds(0, L)] = t1
    tot[pl.ds(L, L)] = t2
    pltpu.sync_copy(tot, part_hbm.at[pl.ds(w * 2 * L, 2 * L)])


def _tc_body(x_ref, o_ref):
    xb = x_ref[0]
    C = xb.shape[0]
    m = jnp.max(xb, axis=0)
    iota = lax.broadcasted_iota(jnp.int32, xb.shape, 0)
    idx = jnp.min(jnp.where(xb == m[None], iota, C), axis=0)
    contrib = jnp.sum(jnp.where(iota == idx[None], xb, 0.0), axis=(1, 2))

    @pl.when(pl.program_id(1) == 0)
    def _():
        o_ref[...] = jnp.zeros_like(o_ref)

    o_ref[0, 0, :] += contrib


def kernel(x):
    B, C, H, W = x.shape
    mesh = plsc.VectorSubcoreMesh(
        core_axis_name="c", subcore_axis_name="s",
        num_cores=NC, num_subcores=NS)
    sc_fn = pl.kernel(
        _sc_body,
        out_type=jax.ShapeDtypeStruct((NW * 2 * L,), jnp.float32),
        mesh=mesh,
        compiler_params=pltpu.CompilerParams(
            needs_layout_passes=False, use_tc_tiling_on_sc=True),
        scratch_types=[
            pltpu.VMEM((C, RB, CB), jnp.float32),
            pltpu.VMEM((C, RB, CB), jnp.float32),
            pltpu.VMEM((C * L,), jnp.float32),
            pltpu.VMEM((2 * L,), jnp.float32),
            pltpu.SemaphoreType.DMA,
            pltpu.SemaphoreType.DMA,
        ],
    )
    parts = sc_fn(x)
    out_tc = pl.pallas_call(
        _tc_body,
        grid=(B, H_TC // HC),
        in_specs=[pl.BlockSpec((1, C, HC, W), lambda b, h: (b, 0, h, 0))],
        out_specs=pl.BlockSpec((1, 1, C), lambda b, h: (b, 0, 0)),
        out_shape=jax.ShapeDtypeStruct((B, 1, C), x.dtype),
    )(x)
    return out_tc.reshape(B, C) + parts.reshape(B, 4, 2 * L).sum(axis=1)[:, :C]


# speedup vs baseline: 1.0200x; 1.0200x over previous
import jax
import jax.numpy as jnp
from jax import lax
from jax.experimental import pallas as pl
from jax.experimental.pallas import tpu as pltpu
from jax.experimental.pallas import tpu_sc as plsc

NC = 2
NS = 16
L = 16
NW = NC * NS
RB = 8
CB = 256
H_TC = 256
HC = 64


def _sc_body(x_hbm, part_hbm, buf0, buf1, acc, tot, sem0, sem1):
    B, C, H, W = x_hbm.shape
    w = lax.axis_index("s") * NC + lax.axis_index("c")
    b = w // 4
    rows_per_w = (H - H_TC) // 4
    row0 = H_TC + (w % 4) * rows_per_w
    n_cb = W // CB
    nchunk = (rows_per_w // RB) * n_cb

    zero = jnp.zeros((L,), jnp.float32)
    for i in range(C):
        acc[pl.ds(i * L, L)] = zero

    lanes = lax.iota(jnp.int32, L)

    def copy(g, buf, sem):
        r = row0 + (g // n_cb) * RB
        col = (g % n_cb) * CB
        return pltpu.make_async_copy(
            x_hbm.at[b, :, pl.ds(r, RB), pl.ds(col, CB)], buf, sem)

    copy(0, buf0, sem0).start()

    def compute(buf):
        def combine(a, b_):
            ma, ia = a
            mb, ib = b_
            gt = mb > ma
            m = jnp.maximum(ma, mb)
            iav = jnp.full((L,), ia, jnp.int32) if isinstance(ia, int) else ia
            ibv = jnp.full((L,), ib, jnp.int32) if isinstance(ib, int) else ib
            return m, jnp.where(gt, ibv, iav)

        NV = CB // L
        def vbody(v, carry):
            i = v // NV
            s = (v % NV) * L
            nodes = [(buf[c, i, pl.ds(s, L)], c) for c in range(C)]
            while len(nodes) > 1:
                nxt = [combine(nodes[k], nodes[k + 1])
                       for k in range(0, len(nodes) - 1, 2)]
                if len(nodes) % 2:
                    nxt.append(nodes[-1])
                nodes = nxt
            m, idx = nodes[0]
            plsc.addupdate_scatter(acc, [idx * L + lanes], m)
            return carry

        lax.fori_loop(0, RB * CB // L, vbody, 0)

    def outer(g2, carry):
        g = g2 * 2
        copy(g, buf0, sem0).wait()
        copy(g + 1, buf1, sem1).start()
        compute(buf0)
        copy(g + 1, buf1, sem1).wait()

        @pl.when(g + 2 < nchunk)
        def _():
            copy(g + 2, buf0, sem0).start()

        compute(buf1)
        return carry

    lax.fori_loop(0, nchunk // 2, outer, 0)

    ci1 = lanes
    ci2 = jnp.minimum(lanes + 16, C - 1)
    t1 = jnp.zeros((L,), jnp.float32)
    t2 = jnp.zeros((L,), jnp.float32)
    for l in range(L):
        t1 = t1 + plsc.load_gather(acc, [ci1 * L + l])
        t2 = t2 + plsc.load_gather(acc, [ci2 * L + l])
    t2 = jnp.where(lanes < C - 16, t2, 0.0)
    tot[pl.ds(0, L)] = t1
    tot[pl.ds(L, L)] = t2
    pltpu.sync_copy(tot, part_hbm.at[pl.ds(w * 2 * L, 2 * L)])


def _tc_body(x_ref, o_ref):
    xb = x_ref[0]
    C = xb.shape[0]
    m = jnp.max(xb, axis=0)
    iota = lax.broadcasted_iota(jnp.int32, xb.shape, 0)
    idx = jnp.min(jnp.where(xb == m[None], iota, C), axis=0)
    contrib = jnp.sum(jnp.where(iota == idx[None], xb, 0.0), axis=(1, 2))

    @pl.when(pl.program_id(1) == 0)
    def _():
        o_ref[...] = jnp.zeros_like(o_ref)

    o_ref[0, 0, :] += contrib


def kernel(x):
    B, C, H, W = x.shape
    mesh = plsc.VectorSubcoreMesh(
        core_axis_name="c", subcore_axis_name="s",
        num_cores=NC, num_subcores=NS)
    sc_fn = pl.kernel(
        _sc_body,
        out_type=jax.ShapeDtypeStruct((NW * 2 * L,), jnp.float32),
        mesh=mesh,
        compiler_params=pltpu.CompilerParams(
            needs_layout_passes=False, use_tc_tiling_on_sc=True),
        scratch_types=[
            pltpu.VMEM((C, RB, CB), jnp.float32),
            pltpu.VMEM((C, RB, CB), jnp.float32),
            pltpu.VMEM((C * L,), jnp.float32),
            pltpu.VMEM((2 * L,), jnp.float32),
            pltpu.SemaphoreType.DMA,
            pltpu.SemaphoreType.DMA,
        ],
    )
    parts = sc_fn(x)
    out_tc = pl.pallas_call(
        _tc_body,
        grid=(B, H_TC // HC),
        in_specs=[pl.BlockSpec((1, C, HC, W), lambda b, h: (b, 0, h, 0))],
        out_specs=pl.BlockSpec((1, 1, C), lambda b, h: (b, 0, 0)),
        out_shape=jax.ShapeDtypeStruct((B, 1, C), x.dtype),
    )(x)
    return out_tc.reshape(B, C) + parts.reshape(B, 4, 2 * L).sum(axis=1)[:, :C]
